# baseline (device time: 44682 ns/iter reference)
import jax
import jax.numpy as jnp
from jax import lax
from jax.experimental import pallas as pl
from jax.experimental.pallas import tpu as pltpu

N_DEV = 4
T = 512
D = 512
F = 1024
NE = 2


def kernel(x, assign, W1, W2):
    a2 = assign.reshape(T, 1)

    def body(x_ref, a_ref, w1_ref, w2_ref, out_ref,
             xs_ref, as_ref, w1b_ref, w2b_ref, contrib_ref, comb_ref,
             x_send, x_recv, a_send, a_recv, c_send, c_recv):
        my = lax.axis_index("i")

        bsem = pltpu.get_barrier_semaphore()
        for o in range(1, N_DEV):
            peer = lax.rem(my + o, N_DEV)
            pl.semaphore_signal(bsem, inc=1, device_id=(peer,),
                                device_id_type=pl.DeviceIdType.MESH)
        pl.semaphore_wait(bsem, N_DEV - 1)

        w1b_ref[...] = w1_ref[...].astype(jnp.bfloat16)
        w2b_ref[...] = w2_ref[...].astype(jnp.bfloat16)
        xs_ref[pl.ds(my * T, T), :] = x_ref[...].astype(jnp.bfloat16)
        as_ref[pl.ds(my * T, T), :] = a_ref[...]

        x_rdmas = []
        a_rdmas = []
        for o in range(1, N_DEV):
            tgt = lax.rem(my + o, N_DEV)
            i = o - 1
            rx = pltpu.make_async_remote_copy(
                src_ref=xs_ref.at[pl.ds(my * T, T), :],
                dst_ref=xs_ref.at[pl.ds(my * T, T), :],
                send_sem=x_send.at[i],
                recv_sem=x_recv.at[i],
                device_id=(tgt,),
                device_id_type=pl.DeviceIdType.MESH,
            )
            ra = pltpu.make_async_remote_copy(
                src_ref=as_ref.at[pl.ds(my * T, T), :],
                dst_ref=as_ref.at[pl.ds(my * T, T), :],
                send_sem=a_send.at[i],
                recv_sem=a_recv.at[i],
                device_id=(tgt,),
                device_id_type=pl.DeviceIdType.MESH,
            )
            rx.start()
            ra.start()
            x_rdmas.append(rx)
            a_rdmas.append(ra)

        def compute_chunk(c):
            a_col = as_ref[pl.ds(c * T, T), :]
            xb = xs_ref[pl.ds(c * T, T), :]
            acc = jnp.zeros((T, D), jnp.float32)
            for l in range(NE):
                eid = my * NE + l
                msk = (a_col == eid).astype(jnp.bfloat16)
                xm = xb * msk
                h = jnp.maximum(
                    jnp.dot(xm, w1b_ref[l], preferred_element_type=jnp.float32),
                    0.0,
                )
                acc = acc + jnp.dot(
                    h.astype(jnp.bfloat16), w2b_ref[l],
                    preferred_element_type=jnp.float32,
                )
            contrib_ref[pl.ds(c * T, T), :] = acc.astype(jnp.bfloat16)

        compute_chunk(my)

        c_rdmas = []
        for i in (0, 2, 1):
            x_rdmas[i].wait()
            a_rdmas[i].wait()
            c = lax.rem(my + N_DEV - 1 - i, N_DEV)
            compute_chunk(c)
            ci = N_DEV - 2 - i
            rc = pltpu.make_async_remote_copy(
                src_ref=contrib_ref.at[pl.ds(c * T, T), :],
                dst_ref=comb_ref.at[pl.ds(ci * T, T), :],
                send_sem=c_send.at[ci],
                recv_sem=c_recv.at[ci],
                device_id=(c,),
                device_id_type=pl.DeviceIdType.MESH,
            )
            rc.start()
            c_rdmas.append(rc)

        tot = contrib_ref[pl.ds(my * T, T), :].astype(jnp.float32)
        for rc in c_rdmas:
            rc.wait()
        for s in range(N_DEV - 1):
            tot = tot + comb_ref[pl.ds(s * T, T), :].astype(jnp.float32)
        out_ref[...] = tot

    return pl.pallas_call(
        body,
        out_shape=jax.ShapeDtypeStruct((T, D), jnp.float32),
        in_specs=[pl.BlockSpec(memory_space=pltpu.VMEM)] * 4,
        out_specs=pl.BlockSpec(memory_space=pltpu.VMEM),
        scratch_shapes=[
            pltpu.VMEM((N_DEV * T, D), jnp.bfloat16),
            pltpu.VMEM((N_DEV * T, 1), jnp.int32),
            pltpu.VMEM((NE, D, F), jnp.bfloat16),
            pltpu.VMEM((NE, F, D), jnp.bfloat16),
            pltpu.VMEM((N_DEV * T, D), jnp.bfloat16),
            pltpu.VMEM(((N_DEV - 1) * T, D), jnp.bfloat16),
            pltpu.SemaphoreType.DMA((N_DEV - 1,)),
            pltpu.SemaphoreType.DMA((N_DEV - 1,)),
            pltpu.SemaphoreType.DMA((N_DEV - 1,)),
            pltpu.SemaphoreType.DMA((N_DEV - 1,)),
            pltpu.SemaphoreType.DMA((N_DEV - 1,)),
            pltpu.SemaphoreType.DMA((N_DEV - 1,)),
        ],
        compiler_params=pltpu.CompilerParams(collective_id=0),
    )(x, a2, W1, W2)


# device time: 41527 ns/iter; 1.0760x vs baseline; 1.0760x over previous
import jax
import jax.numpy as jnp
from jax import lax
from jax.experimental import pallas as pl
from jax.experimental.pallas import tpu as pltpu

N_DEV = 4
T = 512
T2 = T // 2
D = 512
F = 1024
NE = 2


def kernel(x, assign, W1, W2):
    a2 = assign.reshape(T, 1)

    def body(x_ref, a_ref, w1_ref, w2_ref, out_ref,
             xs_ref, as_ref, w1b_ref, w2b_ref, contrib_ref, comb_ref,
             x_send, x_recv, a_send, a_recv, c_send, c_recv):
        my = lax.axis_index("i")

        bsem = pltpu.get_barrier_semaphore()
        for o in range(1, N_DEV):
            peer = lax.rem(my + o, N_DEV)
            pl.semaphore_signal(bsem, inc=1, device_id=(peer,),
                                device_id_type=pl.DeviceIdType.MESH)
        pl.semaphore_wait(bsem, N_DEV - 1)

        xs_ref[pl.ds(my * T, T), :] = x_ref[...].astype(jnp.bfloat16)
        as_ref[pl.ds(my * T, T), :] = a_ref[...]

        a_rdmas = []
        for o in range(1, N_DEV):
            tgt = lax.rem(my + o, N_DEV)
            ra = pltpu.make_async_remote_copy(
                src_ref=as_ref.at[pl.ds(my * T, T), :],
                dst_ref=as_ref.at[pl.ds(my * T, T), :],
                send_sem=a_send.at[o - 1],
                recv_sem=a_recv.at[o - 1],
                device_id=(tgt,),
                device_id_type=pl.DeviceIdType.MESH,
            )
            ra.start()
            a_rdmas.append(ra)
        x_rdmas = {}
        for h in range(2):
            for o in (1, 3, 2):
                tgt = lax.rem(my + o, N_DEV)
                rx = pltpu.make_async_remote_copy(
                    src_ref=xs_ref.at[pl.ds(my * T + h * T2, T2), :],
                    dst_ref=xs_ref.at[pl.ds(my * T + h * T2, T2), :],
                    send_sem=x_send.at[o - 1, h],
                    recv_sem=x_recv.at[o - 1, h],
                    device_id=(tgt,),
                    device_id_type=pl.DeviceIdType.MESH,
                )
                rx.start()
                x_rdmas[(o - 1, h)] = rx

        w1b_ref[...] = w1_ref[...].astype(jnp.bfloat16)
        w2b_ref[...] = w2_ref[...].astype(jnp.bfloat16)

        def compute_half(c, h):
            r0 = c * T + h * T2
            a_col = as_ref[pl.ds(r0, T2), :]
            xb = xs_ref[pl.ds(r0, T2), :]
            acc = jnp.zeros((T2, D), jnp.float32)
            for l in range(NE):
                eid = my * NE + l
                msk = (a_col == eid).astype(jnp.bfloat16)
                xm = xb * msk
                hh = jnp.maximum(
                    jnp.dot(xm, w1b_ref[l], preferred_element_type=jnp.float32),
                    0.0,
                )
                acc = acc + jnp.dot(
                    hh.astype(jnp.bfloat16), w2b_ref[l],
                    preferred_element_type=jnp.float32,
                )
            contrib_ref[pl.ds(r0, T2), :] = acc.astype(jnp.bfloat16)

        compute_half(my, 0)
        compute_half(my, 1)

        c_rdmas = []
        for i in (0, 2, 1):
            a_rdmas[i].wait_recv()
            c = lax.rem(my + N_DEV - 1 - i, N_DEV)
            ci = N_DEV - 2 - i
            for h in range(2):
                x_rdmas[(i, h)].wait_recv()
                compute_half(c, h)
                rc = pltpu.make_async_remote_copy(
                    src_ref=contrib_ref.at[pl.ds(c * T + h * T2, T2), :],
                    dst_ref=comb_ref.at[pl.ds(ci * T + h * T2, T2), :],
                    send_sem=c_send.at[ci, h],
                    recv_sem=c_recv.at[ci, h],
                    device_id=(c,),
                    device_id_type=pl.DeviceIdType.MESH,
                )
                rc.start()
                c_rdmas.append(rc)

        for rc in c_rdmas:
            rc.wait_recv()
        tot = contrib_ref[pl.ds(my * T, T), :].astype(jnp.float32)
        for s in range(N_DEV - 1):
            tot = tot + comb_ref[pl.ds(s * T, T), :].astype(jnp.float32)
        out_ref[...] = tot

        for ra in a_rdmas:
            ra.wait_send()
        for rx in x_rdmas.values():
            rx.wait_send()
        for rc in c_rdmas:
            rc.wait_send()

    return pl.pallas_call(
        body,
        out_shape=jax.ShapeDtypeStruct((T, D), jnp.float32),
        in_specs=[pl.BlockSpec(memory_space=pltpu.VMEM)] * 4,
        out_specs=pl.BlockSpec(memory_space=pltpu.VMEM),
        scratch_shapes=[
            pltpu.VMEM((N_DEV * T, D), jnp.bfloat16),
            pltpu.VMEM((N_DEV * T, 1), jnp.int32),
            pltpu.VMEM((NE, D, F), jnp.bfloat16),
            pltpu.VMEM((NE, F, D), jnp.bfloat16),
            pltpu.VMEM((N_DEV * T, D), jnp.bfloat16),
            pltpu.VMEM(((N_DEV - 1) * T, D), jnp.bfloat16),
            pltpu.SemaphoreType.DMA((N_DEV - 1, 2)),
            pltpu.SemaphoreType.DMA((N_DEV - 1, 2)),
            pltpu.SemaphoreType.DMA((N_DEV - 1,)),
            pltpu.SemaphoreType.DMA((N_DEV - 1,)),
            pltpu.SemaphoreType.DMA((N_DEV - 1, 2)),
            pltpu.SemaphoreType.DMA((N_DEV - 1, 2)),
        ],
        compiler_params=pltpu.CompilerParams(collective_id=0),
    )(x, a2, W1, W2)
